# baseline (speedup 1.0000x reference)
import functools

import jax
import jax.numpy as jnp
from jax import lax
from jax.experimental import pallas as pl
from jax.experimental.pallas import tpu as pltpu
from jax.experimental.pallas import tpu_sc as plsc

_B, _S, _D = 4, 8192, 2048
_NPOS = 64
_BS = _B * _S
_NC, _NS = 2, 16
_NW = _NC * _NS
_ROWS = _B * _NPOS
_RPW = _ROWS // _NW
_NCHUNK = 32
_CHUNK = _BS // _NCHUNK


@functools.cache
def _tc_copy():
    mesh = pltpu.create_tensorcore_mesh("core")
    n_cores = mesh.shape["core"]

    @functools.partial(
        pl.kernel,
        mesh=mesh,
        scratch_types=[pltpu.SemaphoreType.DMA],
    )
    def body(in_hbm, out_hbm, sem):
        core = lax.axis_index("core")
        copies = []
        for j in range(_NCHUNK // n_cores):
            base = (core * (_NCHUNK // n_cores) + j) * _CHUNK
            c = pltpu.make_async_copy(
                in_hbm.at[pl.ds(base, _CHUNK)],
                out_hbm.at[pl.ds(base, _CHUNK)],
                sem,
            )
            c.start()
            copies.append(c)
        for c in copies:
            c.wait()

    return body


@functools.cache
def _sc_scatter():
    @functools.partial(
        pl.kernel,
        mesh=plsc.VectorSubcoreMesh(
            core_axis_name="c", subcore_axis_name="s",
            num_cores=_NC, num_subcores=_NS,
        ),
        scratch_types=[
            pltpu.VMEM((_RPW,), jnp.int32),
            pltpu.VMEM((_RPW, _D), jnp.float32),
            pltpu.SemaphoreType.DMA,
        ],
    )
    def body(act_hbm, idx_hbm, out_hbm, idx_v, act_v, sem):
        w = lax.axis_index("s") * _NC + lax.axis_index("c")
        g = (w * _RPW) % _NPOS
        pltpu.sync_copy(idx_hbm.at[w], idx_v)
        pltpu.sync_copy(act_hbm.at[pl.ds(g, _RPW)], act_v)
        pltpu.async_copy(act_v, out_hbm.at[idx_v], sem).wait()

    return body


def kernel(output, activation, token_position):
    flat = output.reshape(_BS, _D)
    row_idx = (
        token_position[None, :].astype(jnp.int32)
        + (jnp.arange(_B, dtype=jnp.int32) * _S)[:, None]
    ).reshape(_NW, _RPW)
    out_ref = jax.new_ref(lax.empty((_BS, _D), jnp.float32))
    _tc_copy()(flat, out_ref)
    _sc_scatter()(activation, row_idx, out_ref)
    return jax.freeze(out_ref).reshape(_B, _S, _D)


# speedup vs baseline: 43.7886x; 43.7886x over previous
import functools

import jax
import jax.numpy as jnp
from jax import lax
from jax.experimental import pallas as pl
from jax.experimental.pallas import tpu as pltpu
from jax.experimental.pallas import tpu_sc as plsc

_B, _S, _D = 4, 8192, 2048
_NPOS = 64
_BS = _B * _S
_NC, _NS = 2, 16
_NW = _NC * _NS
_ROWS = _B * _NPOS
_RPW = _ROWS // _NW
_NCHUNK = 32
_CHUNK = _BS // _NCHUNK


@functools.cache
def _tc_copy():
    mesh = pltpu.create_tensorcore_mesh("core")

    def blk(i_vmem, o_vmem):
        o_vmem[...] = i_vmem[...]

    @functools.partial(pl.kernel, mesh=mesh)
    def body(in_hbm, out_hbm):
        pltpu.emit_pipeline(
            blk,
            grid=(_NCHUNK,),
            in_specs=[pl.BlockSpec((_CHUNK, _D), lambda i: (i, 0))],
            out_specs=[pl.BlockSpec((_CHUNK, _D), lambda i: (i, 0))],
            core_axis_name="core",
        )(in_hbm, out_hbm)

    return body


@functools.cache
def _sc_scatter():
    @functools.partial(
        pl.kernel,
        mesh=plsc.VectorSubcoreMesh(
            core_axis_name="c", subcore_axis_name="s",
            num_cores=_NC, num_subcores=_NS,
        ),
        scratch_types=[
            pltpu.VMEM((_RPW,), jnp.int32),
            pltpu.VMEM((_RPW, _D), jnp.float32),
            pltpu.SemaphoreType.DMA,
        ],
    )
    def body(act_hbm, idx_hbm, out_hbm, idx_v, act_v, sem):
        w = lax.axis_index("s") * _NC + lax.axis_index("c")
        g = (w * _RPW) % _NPOS
        pltpu.sync_copy(idx_hbm.at[w], idx_v)
        pltpu.sync_copy(act_hbm.at[pl.ds(g, _RPW)], act_v)
        pltpu.async_copy(act_v, out_hbm.at[idx_v], sem).wait()

    return body


def kernel(output, activation, token_position):
    flat = output.reshape(_BS, _D)
    row_idx = (
        token_position[None, :].astype(jnp.int32)
        + (jnp.arange(_B, dtype=jnp.int32) * _S)[:, None]
    ).reshape(_NW, _RPW)
    out_ref = jax.new_ref(lax.empty((_BS, _D), jnp.float32))
    _tc_copy()(flat, out_ref)
    _sc_scatter()(activation, row_idx, out_ref)
    return jax.freeze(out_ref).reshape(_B, _S, _D)


# speedup vs baseline: 43.9868x; 1.0045x over previous
import functools

import jax
import jax.numpy as jnp
from jax import lax
from jax.experimental import pallas as pl
from jax.experimental.pallas import tpu as pltpu
from jax.experimental.pallas import tpu_sc as plsc

_B, _S, _D = 4, 8192, 2048
_NPOS = 64
_BS = _B * _S
_NC, _NS = 2, 16
_NW = _NC * _NS
_ROWS = _B * _NPOS
_RPW = _ROWS // _NW


@functools.cache
def _sc_scatter():
    @functools.partial(
        pl.kernel,
        mesh=plsc.VectorSubcoreMesh(
            core_axis_name="c", subcore_axis_name="s",
            num_cores=_NC, num_subcores=_NS,
        ),
        scratch_types=[
            pltpu.VMEM((_RPW,), jnp.int32),
            pltpu.VMEM((_RPW, _D), jnp.float32),
            pltpu.SemaphoreType.DMA,
            pltpu.SemaphoreType.DMA,
        ],
    )
    def body(act_hbm, idx_hbm, out_hbm, idx_v, act_v, s_idx, s_act):
        w = lax.axis_index("s") * _NC + lax.axis_index("c")
        g = (w * _RPW) % _NPOS
        st_idx = pltpu.make_async_copy(idx_hbm.at[w], idx_v, s_idx)
        st_idx.start()
        st_act = pltpu.make_async_copy(act_hbm.at[pl.ds(g, _RPW)], act_v, s_act)
        st_act.start()
        st_idx.wait()
        st_act.wait()
        pltpu.async_copy(act_v, out_hbm.at[idx_v], s_idx).wait()

    return body


def kernel(output, activation, token_position):
    flat = output.reshape(_BS, _D)
    row_idx = (
        token_position[None, :].astype(jnp.int32)
        + (jnp.arange(_B, dtype=jnp.int32) * _S)[:, None]
    ).reshape(_NW, _RPW)
    out_ref = jax.new_ref(flat)
    _sc_scatter()(activation, row_idx, out_ref)
    return jax.freeze(out_ref).reshape(_B, _S, _D)
